# baseline (speedup 1.0000x reference)
import jax
import jax.numpy as jnp
from jax.experimental import pallas as pl
from jax.experimental.pallas import tpu as pltpu

TOP_K = 8
N_EXPERTS = 64
HIDDEN = 2048
ROW_BLOCK = 2048


def _gate_body(x_ref, wt_ref, b_ref, idx_ref, w_ref, lt_ref):
    i = pl.program_id(0)
    nb = pl.num_programs(0) - 1

    @pl.when(i < nb)
    def _matmul():
        xb = x_ref[...].astype(jnp.bfloat16)
        logits = (
            jnp.dot(xb, wt_ref[...], preferred_element_type=jnp.float32)
            + b_ref[...]
        )
        lt_ref[i % 2] = logits.T

    @pl.when(i > 0)
    def _topk():
        lt = lt_ref[(i - 1) % 2]
        rows = lt.shape[1]
        expert_iota = jax.lax.broadcasted_iota(
            jnp.int32, (N_EXPERTS, rows), 0)

        vals = []
        idxs = []
        cur = lt
        for _ in range(TOP_K):
            m = jnp.max(cur, axis=0, keepdims=True)
            hit = cur == m
            sel = jnp.min(jnp.where(hit, expert_iota, N_EXPERTS), axis=0,
                          keepdims=True)
            vals.append(m)
            idxs.append(sel)
            cur = jnp.where(expert_iota == sel, -jnp.inf, cur)

        v = jnp.concatenate(vals, axis=0)
        ii = jnp.concatenate(idxs, axis=0)
        e = jnp.exp(v - v[0:1])
        w = e / jnp.sum(e, axis=0, keepdims=True)
        idx_ref[...] = ii.T
        w_ref[...] = w.T


def kernel(hidden_states, weight, e_score_correction_bias):
    x = hidden_states.reshape(-1, HIDDEN)
    n_rows = x.shape[0]
    wt = weight.T.astype(jnp.bfloat16)
    b = e_score_correction_bias.reshape(1, N_EXPERTS)

    nb = n_rows // ROW_BLOCK
    grid = (nb + 1,)
    idx, w = pl.pallas_call(
        _gate_body,
        grid=grid,
        in_specs=[
            pl.BlockSpec((ROW_BLOCK, HIDDEN),
                         lambda i: (jnp.minimum(i, nb - 1), 0)),
            pl.BlockSpec((HIDDEN, N_EXPERTS), lambda i: (0, 0)),
            pl.BlockSpec((1, N_EXPERTS), lambda i: (0, 0)),
        ],
        out_specs=[
            pl.BlockSpec((ROW_BLOCK, TOP_K),
                         lambda i: (jnp.maximum(i - 1, 0), 0)),
            pl.BlockSpec((ROW_BLOCK, TOP_K),
                         lambda i: (jnp.maximum(i - 1, 0), 0)),
        ],
        out_shape=[
            jax.ShapeDtypeStruct((n_rows, TOP_K), jnp.int32),
            jax.ShapeDtypeStruct((n_rows, TOP_K), jnp.float32),
        ],
        scratch_shapes=[
            pltpu.VMEM((2, N_EXPERTS, ROW_BLOCK), jnp.float32),
        ],
        compiler_params=pltpu.CompilerParams(
            dimension_semantics=("arbitrary",),
        ),
    )(x, wt, b)
    return idx, w


# speedup vs baseline: 1.0430x; 1.0430x over previous
import functools

import jax
import jax.numpy as jnp
from jax.experimental import pallas as pl
from jax.experimental.pallas import tpu as pltpu

TOP_K = 8
N_EXPERTS = 64
HIDDEN = 2048
ROW_BLOCK = 2048


def _gate_body(x_ref, wt_ref, b_ref, idx_ref, w_ref):
    xb = x_ref[...].astype(jnp.bfloat16)
    logits = jnp.dot(xb, wt_ref[...], preferred_element_type=jnp.float32)
    logits = logits + b_ref[...]

    lt = logits.T
    rows = lt.shape[1]
    expert_iota = jax.lax.broadcasted_iota(jnp.int32, (N_EXPERTS, rows), 0)

    vals = []
    idxs = []
    cur = lt
    for _ in range(TOP_K):
        m = jnp.max(cur, axis=0, keepdims=True)
        hit = cur == m
        sel = jnp.min(jnp.where(hit, expert_iota, N_EXPERTS), axis=0,
                      keepdims=True)
        vals.append(m)
        idxs.append(sel)
        cur = jnp.where(expert_iota == sel, -jnp.inf, cur)

    v = jnp.concatenate(vals, axis=0)
    i = jnp.concatenate(idxs, axis=0)
    e = jnp.exp(v - v[0:1])
    w = e / jnp.sum(e, axis=0, keepdims=True)
    idx_ref[...] = i.T
    w_ref[...] = w.T


def kernel(hidden_states, weight, e_score_correction_bias):
    x = hidden_states.reshape(-1, HIDDEN)
    n_rows = x.shape[0]
    wt = weight.T.astype(jnp.bfloat16)
    b = e_score_correction_bias.reshape(1, N_EXPERTS)

    grid = (n_rows // ROW_BLOCK,)
    idx, w = pl.pallas_call(
        _gate_body,
        grid=grid,
        in_specs=[
            pl.BlockSpec((ROW_BLOCK, HIDDEN), lambda i: (i, 0)),
            pl.BlockSpec((HIDDEN, N_EXPERTS), lambda i: (0, 0)),
            pl.BlockSpec((1, N_EXPERTS), lambda i: (0, 0)),
        ],
        out_specs=[
            pl.BlockSpec((ROW_BLOCK, TOP_K), lambda i: (i, 0)),
            pl.BlockSpec((ROW_BLOCK, TOP_K), lambda i: (i, 0)),
        ],
        out_shape=[
            jax.ShapeDtypeStruct((n_rows, TOP_K), jnp.int32),
            jax.ShapeDtypeStruct((n_rows, TOP_K), jnp.float32),
        ],
        compiler_params=pltpu.CompilerParams(
            dimension_semantics=("parallel",),
        ),
    )(x, wt, b)
    return idx, w
